# baseline (speedup 1.0000x reference)
import functools

import jax
import jax.numpy as jnp
from jax import lax
from jax.experimental import pallas as pl
from jax.experimental.pallas import tpu as pltpu
from jax.experimental.pallas import tpu_sc as plsc

EMB = 32
B = 4096
L = 200
N = B * L
NW = 32
BW = B // NW
LB = 2
WORDS = LB * BW
NBLK = L // LB
NBUF = 4
NOUTER = NBLK // NBUF


def _make_gather():
    mesh = plsc.VectorSubcoreMesh(core_axis_name="c", subcore_axis_name="s")

    scratch = (
        [pltpu.VMEM((WORDS,), jnp.int32) for _ in range(NBUF)]
        + [pltpu.VMEM((WORDS, EMB), jnp.float32) for _ in range(NBUF)]
        + [pltpu.VMEM((LB, EMB, BW), jnp.float32) for _ in range(NBUF)]
        + [pltpu.SemaphoreType.DMA for _ in range(3 * NBUF)]
    )

    @functools.partial(
        pl.kernel,
        mesh=mesh,
        out_type=jax.ShapeDtypeStruct((L, EMB, B), jnp.float32),
        scratch_types=scratch,
        compiler_params=pltpu.CompilerParams(
            use_tc_tiling_on_sc=False, needs_layout_passes=False),
    )
    def gather_kernel(ids_hbm, table_hbm, out_hbm, *scratch_refs):
        idx_v = scratch_refs[:NBUF]
        rows_v = scratch_refs[NBUF:2 * NBUF]
        out_v = scratch_refs[2 * NBUF:3 * NBUF]
        isem = scratch_refs[3 * NBUF:4 * NBUF]
        gsem = scratch_refs[4 * NBUF:5 * NBUF]
        wsem = scratch_refs[5 * NBUF:6 * NBUF]

        wid = lax.axis_index("s") * 2 + lax.axis_index("c")
        boff = wid * BW
        iota = lax.iota(jnp.int32, 16)
        zero16 = iota * 0

        def idx_start(b, u):
            for lp in range(LB):
                pltpu.async_copy(
                    ids_hbm.at[u * LB + lp, pl.ds(boff, BW)],
                    idx_v[b].at[pl.ds(lp * BW, BW)], isem[b])

        def idx_wait(b, u):
            for lp in range(LB):
                pltpu.make_async_copy(
                    ids_hbm.at[u * LB + lp, pl.ds(boff, BW)],
                    idx_v[b].at[pl.ds(lp * BW, BW)], isem[b]).wait()

        def gather_start(b):
            pltpu.async_copy(table_hbm.at[idx_v[b]], rows_v[b], gsem[b])

        def gather_wait(b):
            pltpu.make_async_copy(
                table_hbm.at[idx_v[b]], rows_v[b], gsem[b]).wait()

        def extract(b):
            def q_body(q, carry):
                lp = q >> 3
                kg = q & 7
                rows = lp * BW + kg * 16 + iota
                for eh in range(0, EMB, 16):
                    vals = [
                        plsc.load_gather(rows_v[b], [rows, zero16 + (eh + e)])
                        for e in range(16)
                    ]
                    for e in range(16):
                        out_v[b][lp, eh + e, pl.ds(kg * 16, 16)] = vals[e]
                return carry

            lax.fori_loop(0, LB * (BW // 16), q_body, 0)

        def write_start(b, u):
            pltpu.async_copy(
                out_v[b],
                out_hbm.at[pl.ds(u * LB, LB), :, pl.ds(boff, BW)],
                wsem[b])

        def write_wait(b, u):
            pltpu.make_async_copy(
                out_v[b],
                out_hbm.at[pl.ds(u * LB, LB), :, pl.ds(boff, BW)],
                wsem[b]).wait()

        for b in range(NBUF):
            idx_start(b, b)
        for b in range(NBUF):
            idx_wait(b, b)
            gather_start(b)
        for b in range(NBUF):
            gather_wait(b)
            idx_start(b, b + NBUF)
            extract(b)
            write_start(b, b)
            idx_wait(b, b + NBUF)
            gather_start(b)

        def outer(g, carry):
            for b in range(NBUF):
                u = g * NBUF + b
                gather_wait(b)
                idx_start(b, u + NBUF)
                write_wait(b, u - NBUF)
                extract(b)
                write_start(b, u)
                idx_wait(b, u + NBUF)
                gather_start(b)
            return carry

        lax.fori_loop(1, NOUTER - 1, outer, 0)

        last = (NOUTER - 1) * NBUF
        for b in range(NBUF):
            gather_wait(b)
            write_wait(b, last + b - NBUF)
            extract(b)
            write_start(b, last + b)
        for b in range(NBUF):
            write_wait(b, last + b)

    return gather_kernel


_gather_sc = _make_gather()


def kernel(word_ids, table):
    ids_t = word_ids.T
    out = _gather_sc(ids_t, table)
    return out.transpose(2, 0, 1)


# speedup vs baseline: 1.0088x; 1.0088x over previous
import functools

import jax
import jax.numpy as jnp
from jax import lax
from jax.experimental import pallas as pl
from jax.experimental.pallas import tpu as pltpu
from jax.experimental.pallas import tpu_sc as plsc

EMB = 32
B = 4096
L = 200
N = B * L
NW = 32
BW = B // NW
LB = 2
WORDS = LB * BW
NBLK = L // LB
NBUF = 4
NOUTER = NBLK // NBUF


def _make_gather():
    mesh = plsc.VectorSubcoreMesh(core_axis_name="c", subcore_axis_name="s")

    scratch = (
        [pltpu.VMEM((WORDS,), jnp.int32) for _ in range(NBUF)]
        + [pltpu.VMEM((WORDS, EMB), jnp.float32) for _ in range(NBUF)]
        + [pltpu.VMEM((LB, EMB, BW), jnp.float32) for _ in range(NBUF)]
        + [pltpu.SemaphoreType.DMA for _ in range(3 * NBUF)]
    )

    @functools.partial(
        pl.kernel,
        mesh=mesh,
        out_type=jax.ShapeDtypeStruct((L, EMB, B), jnp.float32),
        scratch_types=scratch,
        compiler_params=pltpu.CompilerParams(
            use_tc_tiling_on_sc=False, needs_layout_passes=False),
    )
    def gather_kernel(ids_hbm, table_hbm, out_hbm, *scratch_refs):
        idx_v = scratch_refs[:NBUF]
        rows_v = scratch_refs[NBUF:2 * NBUF]
        out_v = scratch_refs[2 * NBUF:3 * NBUF]
        isem = scratch_refs[3 * NBUF:4 * NBUF]
        gsem = scratch_refs[4 * NBUF:5 * NBUF]
        wsem = scratch_refs[5 * NBUF:6 * NBUF]

        wid = lax.axis_index("s") * 2 + lax.axis_index("c")
        boff = wid * BW
        iota = lax.iota(jnp.int32, 16)
        zero16 = iota * 0

        def idx_start(b, u):
            for lp in range(LB):
                pltpu.async_copy(
                    ids_hbm.at[u * LB + lp, pl.ds(boff, BW)],
                    idx_v[b].at[pl.ds(lp * BW, BW)], isem[b])

        def idx_wait(b, u):
            for lp in range(LB):
                pltpu.make_async_copy(
                    ids_hbm.at[u * LB + lp, pl.ds(boff, BW)],
                    idx_v[b].at[pl.ds(lp * BW, BW)], isem[b]).wait()

        def gather_start(b):
            pltpu.async_copy(table_hbm.at[idx_v[b]], rows_v[b], gsem[b])

        def gather_wait(b):
            pltpu.make_async_copy(
                table_hbm.at[idx_v[b]], rows_v[b], gsem[b]).wait()

        def extract(b):
            def q_body(q, carry):
                lp = q >> 3
                kg = q & 7
                rows = lp * BW + kg * 16 + iota
                vals = [
                    plsc.load_gather(rows_v[b], [rows, zero16 + e])
                    for e in range(EMB)
                ]
                for e in range(EMB):
                    out_v[b][lp, e, pl.ds(kg * 16, 16)] = vals[e]
                return carry

            lax.fori_loop(0, LB * (BW // 16), q_body, 0)

        def write_start(b, u):
            pltpu.async_copy(
                out_v[b],
                out_hbm.at[pl.ds(u * LB, LB), :, pl.ds(boff, BW)],
                wsem[b])

        def write_wait(b, u):
            pltpu.make_async_copy(
                out_v[b],
                out_hbm.at[pl.ds(u * LB, LB), :, pl.ds(boff, BW)],
                wsem[b]).wait()

        for b in range(NBUF):
            idx_start(b, b)
        for b in range(NBUF):
            idx_wait(b, b)
            gather_start(b)
        for b in range(NBUF):
            gather_wait(b)
            idx_start(b, b + NBUF)
            extract(b)
            write_start(b, b)
            idx_wait(b, b + NBUF)
            gather_start(b)

        def outer(g, carry):
            for b in range(NBUF):
                u = g * NBUF + b
                gather_wait(b)
                idx_start(b, u + NBUF)
                write_wait(b, u - NBUF)
                extract(b)
                write_start(b, u)
                idx_wait(b, u + NBUF)
                gather_start(b)
            return carry

        lax.fori_loop(1, NOUTER - 1, outer, 0)

        last = (NOUTER - 1) * NBUF
        for b in range(NBUF):
            gather_wait(b)
            write_wait(b, last + b - NBUF)
            extract(b)
            write_start(b, last + b)
        for b in range(NBUF):
            write_wait(b, last + b)

    return gather_kernel


_gather_sc = _make_gather()


def kernel(word_ids, table):
    ids_t = word_ids.T
    out = _gather_sc(ids_t, table)
    return out.transpose(2, 0, 1)


# speedup vs baseline: 1.1731x; 1.1629x over previous
import functools

import jax
import jax.numpy as jnp
from jax import lax
from jax.experimental import pallas as pl
from jax.experimental.pallas import tpu as pltpu
from jax.experimental.pallas import tpu_sc as plsc

EMB = 32
B = 4096
L = 200
N = B * L
NW = 32
BW = B // NW
LB = 2
WORDS = LB * BW
NBLK = L // LB
NBUF = 4
NOUTER = NBLK // NBUF


def _make_gather():
    mesh = plsc.VectorSubcoreMesh(core_axis_name="c", subcore_axis_name="s")

    scratch = (
        [pltpu.VMEM((WORDS,), jnp.int32) for _ in range(NBUF)]
        + [pltpu.VMEM((WORDS, EMB), jnp.float32) for _ in range(NBUF)]
        + [pltpu.VMEM((LB, EMB, BW + 1), jnp.float32) for _ in range(NBUF)]
        + [pltpu.SemaphoreType.DMA for _ in range(3 * NBUF)]
    )

    @functools.partial(
        pl.kernel,
        mesh=mesh,
        out_type=jax.ShapeDtypeStruct((L, EMB, B), jnp.float32),
        scratch_types=scratch,
        compiler_params=pltpu.CompilerParams(
            use_tc_tiling_on_sc=False, needs_layout_passes=False),
    )
    def gather_kernel(ids_hbm, table_hbm, out_hbm, *scratch_refs):
        idx_v = scratch_refs[:NBUF]
        rows_v = scratch_refs[NBUF:2 * NBUF]
        out_v = scratch_refs[2 * NBUF:3 * NBUF]
        isem = scratch_refs[3 * NBUF:4 * NBUF]
        gsem = scratch_refs[4 * NBUF:5 * NBUF]
        wsem = scratch_refs[5 * NBUF:6 * NBUF]

        wid = lax.axis_index("s") * 2 + lax.axis_index("c")
        boff = wid * BW
        iota = lax.iota(jnp.int32, 16)
        zero16 = iota * 0

        def idx_start(b, u):
            for lp in range(LB):
                pltpu.async_copy(
                    ids_hbm.at[u * LB + lp, pl.ds(boff, BW)],
                    idx_v[b].at[pl.ds(lp * BW, BW)], isem[b])

        def idx_wait(b, u):
            for lp in range(LB):
                pltpu.make_async_copy(
                    ids_hbm.at[u * LB + lp, pl.ds(boff, BW)],
                    idx_v[b].at[pl.ds(lp * BW, BW)], isem[b]).wait()

        def gather_start(b):
            pltpu.async_copy(table_hbm.at[idx_v[b]], rows_v[b], gsem[b])

        def gather_wait(b):
            pltpu.make_async_copy(
                table_hbm.at[idx_v[b]], rows_v[b], gsem[b]).wait()

        def extract(b):
            def k_body(k, carry):
                lp = k >> 7
                kk = k & (BW - 1)
                kvec = zero16 + kk
                lvec = zero16 + lp
                for eh in range(0, EMB, 16):
                    val = rows_v[b][k, pl.ds(eh, 16)]
                    plsc.store_scatter(
                        out_v[b], [lvec, iota + eh, kvec], val)
                return carry

            lax.fori_loop(0, WORDS, k_body, 0, unroll=4)

        def write_start(b, u):
            pltpu.async_copy(
                out_v[b].at[:, :, pl.ds(0, BW)],
                out_hbm.at[pl.ds(u * LB, LB), :, pl.ds(boff, BW)],
                wsem[b])

        def write_wait(b, u):
            pltpu.make_async_copy(
                out_v[b].at[:, :, pl.ds(0, BW)],
                out_hbm.at[pl.ds(u * LB, LB), :, pl.ds(boff, BW)],
                wsem[b]).wait()

        for b in range(NBUF):
            idx_start(b, b)
        for b in range(NBUF):
            idx_wait(b, b)
            gather_start(b)
        for b in range(NBUF):
            gather_wait(b)
            idx_start(b, b + NBUF)
            extract(b)
            write_start(b, b)
            idx_wait(b, b + NBUF)
            gather_start(b)

        def outer(g, carry):
            for b in range(NBUF):
                u = g * NBUF + b
                gather_wait(b)
                idx_start(b, u + NBUF)
                write_wait(b, u - NBUF)
                extract(b)
                write_start(b, u)
                idx_wait(b, u + NBUF)
                gather_start(b)
            return carry

        lax.fori_loop(1, NOUTER - 1, outer, 0)

        last = (NOUTER - 1) * NBUF
        for b in range(NBUF):
            gather_wait(b)
            write_wait(b, last + b - NBUF)
            extract(b)
            write_start(b, last + b)
        for b in range(NBUF):
            write_wait(b, last + b)

    return gather_kernel


_gather_sc = _make_gather()


def kernel(word_ids, table):
    ids_t = word_ids.T
    out = _gather_sc(ids_t, table)
    return out.transpose(2, 0, 1)


# speedup vs baseline: 1.1808x; 1.0066x over previous
import functools

import jax
import jax.numpy as jnp
from jax import lax
from jax.experimental import pallas as pl
from jax.experimental.pallas import tpu as pltpu
from jax.experimental.pallas import tpu_sc as plsc

EMB = 32
B = 4096
L = 200
N = B * L
NW = 32
BW = B // NW
LB = 2
WORDS = LB * BW
NBLK = L // LB
NBUF = 4
NOUTER = NBLK // NBUF


def _make_gather():
    mesh = plsc.VectorSubcoreMesh(core_axis_name="c", subcore_axis_name="s")

    scratch = (
        [pltpu.VMEM((WORDS,), jnp.int32) for _ in range(NBUF)]
        + [pltpu.VMEM((WORDS, EMB), jnp.float32) for _ in range(NBUF)]
        + [pltpu.VMEM((LB, EMB, BW + 1), jnp.float32) for _ in range(NBUF)]
        + [pltpu.SemaphoreType.DMA for _ in range(3 * NBUF)]
    )

    @functools.partial(
        pl.kernel,
        mesh=mesh,
        out_type=jax.ShapeDtypeStruct((L, EMB, B), jnp.float32),
        scratch_types=scratch,
        compiler_params=pltpu.CompilerParams(
            use_tc_tiling_on_sc=False, needs_layout_passes=False),
    )
    def gather_kernel(ids_hbm, table_hbm, out_hbm, *scratch_refs):
        idx_v = scratch_refs[:NBUF]
        rows_v = scratch_refs[NBUF:2 * NBUF]
        out_v = scratch_refs[2 * NBUF:3 * NBUF]
        isem = scratch_refs[3 * NBUF:4 * NBUF]
        gsem = scratch_refs[4 * NBUF:5 * NBUF]
        wsem = scratch_refs[5 * NBUF:6 * NBUF]

        wid = lax.axis_index("s") * 2 + lax.axis_index("c")
        boff = wid * BW
        iota = lax.iota(jnp.int32, 16)
        zero16 = iota * 0

        def idx_start(b, u):
            for lp in range(LB):
                pltpu.async_copy(
                    ids_hbm.at[u * LB + lp, pl.ds(boff, BW)],
                    idx_v[b].at[pl.ds(lp * BW, BW)], isem[b])

        def idx_wait(b, u):
            for lp in range(LB):
                pltpu.make_async_copy(
                    ids_hbm.at[u * LB + lp, pl.ds(boff, BW)],
                    idx_v[b].at[pl.ds(lp * BW, BW)], isem[b]).wait()

        def gather_start(b):
            pltpu.async_copy(table_hbm.at[idx_v[b]], rows_v[b], gsem[b])

        def gather_wait(b):
            pltpu.make_async_copy(
                table_hbm.at[idx_v[b]], rows_v[b], gsem[b]).wait()

        def extract(b):
            for lp in range(LB):
                lvec = zero16 + lp

                def kk_body(kk, carry, lp=lp, lvec=lvec):
                    kvec = zero16 + kk
                    for eh in range(0, EMB, 16):
                        val = rows_v[b][lp * BW + kk, pl.ds(eh, 16)]
                        plsc.store_scatter(
                            out_v[b], [lvec, iota + eh, kvec], val)
                    return carry

                lax.fori_loop(0, BW, kk_body, 0, unroll=8)

        def write_start(b, u):
            pltpu.async_copy(
                out_v[b].at[:, :, pl.ds(0, BW)],
                out_hbm.at[pl.ds(u * LB, LB), :, pl.ds(boff, BW)],
                wsem[b])

        def write_wait(b, u):
            pltpu.make_async_copy(
                out_v[b].at[:, :, pl.ds(0, BW)],
                out_hbm.at[pl.ds(u * LB, LB), :, pl.ds(boff, BW)],
                wsem[b]).wait()

        for b in range(NBUF):
            idx_start(b, b)
        for b in range(NBUF):
            idx_wait(b, b)
            gather_start(b)
        for b in range(NBUF):
            gather_wait(b)
            idx_start(b, b + NBUF)
            extract(b)
            write_start(b, b)
            idx_wait(b, b + NBUF)
            gather_start(b)

        def outer(g, carry):
            for b in range(NBUF):
                u = g * NBUF + b
                gather_wait(b)
                idx_start(b, u + NBUF)
                write_wait(b, u - NBUF)
                extract(b)
                write_start(b, u)
                idx_wait(b, u + NBUF)
                gather_start(b)
            return carry

        lax.fori_loop(1, NOUTER - 1, outer, 0)

        last = (NOUTER - 1) * NBUF
        for b in range(NBUF):
            gather_wait(b)
            write_wait(b, last + b - NBUF)
            extract(b)
            write_start(b, last + b)
        for b in range(NBUF):
            write_wait(b, last + b)

    return gather_kernel


_gather_sc = _make_gather()


def kernel(word_ids, table):
    ids_t = word_ids.T
    out = _gather_sc(ids_t, table)
    return out.transpose(2, 0, 1)
